# Initial kernel scaffold; baseline (speedup 1.0000x reference)
import jax
import jax.numpy as jnp
from jax.experimental import pallas as pl


def kernel(x, table):
    raise NotImplementedError("write your pallas kernel here")


# speedup vs baseline: 5.2366x; 5.2366x over previous
import functools

import jax
import jax.numpy as jnp
from jax import lax
from jax.experimental import pallas as pl
from jax.experimental.pallas import tpu as pltpu
from jax.experimental.pallas import tpu_sc as plsc

EMB = 128
CHUNK = 128


def _emb_kernel(n_total, table_hbm, idx_hbm, out_hbm, idx_v, rows_v, sem):
    nc = lax.axis_size("c")
    wid = lax.axis_index("s") * nc + lax.axis_index("c")
    nw = nc * lax.axis_size("s")
    per_w = n_total // nw
    base = wid * per_w
    nchunks = per_w // CHUNK

    def body(i, carry):
        off = base + i * CHUNK
        pltpu.sync_copy(idx_hbm.at[pl.ds(off, CHUNK)], idx_v)
        pltpu.async_copy(table_hbm.at[idx_v], rows_v, sem).wait()
        pltpu.sync_copy(rows_v, out_hbm.at[pl.ds(off, CHUNK)])
        return carry

    lax.fori_loop(0, nchunks, body, 0)


@jax.jit
def kernel(x, table):
    batch, hist = x.shape
    n_total = batch * hist
    idx = x.reshape(n_total)
    mesh = plsc.VectorSubcoreMesh(core_axis_name="c", subcore_axis_name="s")
    out = pl.kernel(
        functools.partial(_emb_kernel, n_total),
        out_type=jax.ShapeDtypeStruct((n_total, EMB), jnp.float32),
        mesh=mesh,
        scratch_types=[
            pltpu.VMEM((CHUNK,), jnp.int32),
            pltpu.VMEM((CHUNK, EMB), jnp.float32),
            pltpu.SemaphoreType.DMA,
        ],
    )(table, idx)
    return out.reshape(batch, hist, EMB)


# speedup vs baseline: 9.3306x; 1.7818x over previous
import functools

import jax
import jax.numpy as jnp
from jax import lax
from jax.experimental import pallas as pl
from jax.experimental.pallas import tpu as pltpu
from jax.experimental.pallas import tpu_sc as plsc

EMB = 128
CHUNK = 128


def _emb_kernel(n_total, table_hbm, idx_hbm, out_hbm,
                idx_all, rows_a, rows_b, sem_a, sem_b):
    nc = lax.axis_size("c")
    wid = lax.axis_index("s") * nc + lax.axis_index("c")
    nw = nc * lax.axis_size("s")
    per_w = n_total // nw
    nchunks = per_w // CHUNK
    base = wid * per_w
    row_base = wid * nchunks

    pltpu.sync_copy(idx_hbm.at[pl.ds(row_base, nchunks)], idx_all)

    def fire(c, rows, sem):
        pltpu.async_copy(table_hbm.at[idx_all.at[c]], rows, sem)

    def drain(rows, sem):
        pltpu.make_async_copy(table_hbm.at[pl.ds(0, CHUNK)], rows, sem).wait()

    def store(c, rows):
        pltpu.sync_copy(rows, out_hbm.at[pl.ds(base + c * CHUNK, CHUNK)])

    fire(0, rows_a, sem_a)

    def body(j, carry):
        c0 = 2 * j
        fire(c0 + 1, rows_b, sem_b)
        drain(rows_a, sem_a)
        store(c0, rows_a)
        fire(c0 + 2, rows_a, sem_a)
        drain(rows_b, sem_b)
        store(c0 + 1, rows_b)
        return carry

    lax.fori_loop(0, nchunks // 2 - 1, body, 0)

    last = nchunks - 2
    fire(last + 1, rows_b, sem_b)
    drain(rows_a, sem_a)
    store(last, rows_a)
    drain(rows_b, sem_b)
    store(last + 1, rows_b)


@jax.jit
def kernel(x, table):
    batch, hist = x.shape
    n_total = batch * hist
    idx2d = x.reshape(n_total // CHUNK, CHUNK)
    mesh = plsc.VectorSubcoreMesh(core_axis_name="c", subcore_axis_name="s")
    n_sub = 32
    nchunks_w = n_total // n_sub // CHUNK
    out = pl.kernel(
        functools.partial(_emb_kernel, n_total),
        out_type=jax.ShapeDtypeStruct((n_total, EMB), jnp.float32),
        mesh=mesh,
        scratch_types=[
            pltpu.VMEM((nchunks_w, CHUNK), jnp.int32),
            pltpu.VMEM((CHUNK, EMB), jnp.float32),
            pltpu.VMEM((CHUNK, EMB), jnp.float32),
            pltpu.SemaphoreType.DMA,
            pltpu.SemaphoreType.DMA,
        ],
    )(table, idx2d)
    return out.reshape(batch, hist, EMB)


# speedup vs baseline: 9.3721x; 1.0044x over previous
import functools

import jax
import jax.numpy as jnp
from jax import lax
from jax.experimental import pallas as pl
from jax.experimental.pallas import tpu as pltpu
from jax.experimental.pallas import tpu_sc as plsc

EMB = 128
CHUNK = 128
NBUF = 4


def _emb_kernel(n_total, table_hbm, idx_hbm, out_hbm,
                idx_all, r0, r1, r2, r3, g0, g1, g2, g3, s0, s1, s2, s3):
    rows = (r0, r1, r2, r3)
    gsem = (g0, g1, g2, g3)
    ssem = (s0, s1, s2, s3)

    nc = lax.axis_size("c")
    wid = lax.axis_index("s") * nc + lax.axis_index("c")
    nw = nc * lax.axis_size("s")
    per_w = n_total // nw
    nchunks = per_w // CHUNK
    base = wid * per_w
    row_base = wid * nchunks

    pltpu.sync_copy(idx_hbm.at[pl.ds(row_base, nchunks)], idx_all)

    def fire_g(c, b):
        pltpu.async_copy(table_hbm.at[idx_all.at[c]], rows[b], gsem[b])

    def drain_g(b):
        pltpu.make_async_copy(
            table_hbm.at[pl.ds(0, CHUNK)], rows[b], gsem[b]).wait()

    def fire_s(c, b):
        pltpu.async_copy(
            rows[b], out_hbm.at[pl.ds(base + c * CHUNK, CHUNK)], ssem[b])

    def drain_s(b):
        pltpu.make_async_copy(
            rows[b], out_hbm.at[pl.ds(base, CHUNK)], ssem[b]).wait()

    for b in range(NBUF):
        fire_g(b, b)

    def body(j, carry):
        c = NBUF * j
        drain_g(0); fire_s(c + 0, 0)
        drain_g(1); fire_s(c + 1, 1)
        drain_s(0); fire_g(c + 4, 0)
        drain_g(2); fire_s(c + 2, 2)
        drain_s(1); fire_g(c + 5, 1)
        drain_g(3); fire_s(c + 3, 3)
        drain_s(2); fire_g(c + 6, 2)
        drain_s(3); fire_g(c + 7, 3)
        return carry

    lax.fori_loop(0, nchunks // NBUF - 1, body, 0)

    last = nchunks - NBUF
    for b in range(NBUF):
        drain_g(b)
        fire_s(last + b, b)
    for b in range(NBUF):
        drain_s(b)


@jax.jit
def kernel(x, table):
    batch, hist = x.shape
    n_total = batch * hist
    idx2d = x.reshape(n_total // CHUNK, CHUNK)
    mesh = plsc.VectorSubcoreMesh(core_axis_name="c", subcore_axis_name="s")
    n_sub = 32
    nchunks_w = n_total // n_sub // CHUNK
    out = pl.kernel(
        functools.partial(_emb_kernel, n_total),
        out_type=jax.ShapeDtypeStruct((n_total, EMB), jnp.float32),
        mesh=mesh,
        scratch_types=(
            [pltpu.VMEM((nchunks_w, CHUNK), jnp.int32)]
            + [pltpu.VMEM((CHUNK, EMB), jnp.float32)] * NBUF
            + [pltpu.SemaphoreType.DMA] * (2 * NBUF)
        ),
    )(table, idx2d)
    return out.reshape(batch, hist, EMB)
